# baseline (speedup 1.0000x reference)
import functools

import jax
import jax.numpy as jnp
from jax import lax
from jax.experimental import pallas as pl
from jax.experimental.pallas import tpu as pltpu
from jax.experimental.pallas import tpu_sc as plsc

_T = 1000
_B = 16384
_L = 16

_info = plsc.get_sparse_core_info()
_NC = _info.num_cores
_NS = _info.num_subcores
_NW = _NC * _NS
_BPW = _B // _NW
_NIT = _BPW // _L

_mesh = plsc.VectorSubcoreMesh(core_axis_name="c", subcore_axis_name="s")


@functools.partial(
    pl.kernel,
    mesh=_mesh,
    out_type=(
        jax.ShapeDtypeStruct((_B,), jnp.float32),
        jax.ShapeDtypeStruct((_B,), jnp.float32),
        jax.ShapeDtypeStruct((_B,), jnp.float32),
    ),
    compiler_params=pltpu.CompilerParams(
        needs_layout_passes=False,
        disable_bounds_checks=True,
        disable_semaphore_checks=True,
        skip_device_barrier=True,
    ),
    scratch_types=[
        pltpu.VMEM((_T,), jnp.float32),
        pltpu.VMEM((_T,), jnp.float32),
        pltpu.VMEM((_T,), jnp.float32),
        pltpu.VMEM((_BPW,), jnp.int32),
        pltpu.VMEM((_BPW,), jnp.float32),
        pltpu.VMEM((_BPW,), jnp.float32),
        pltpu.VMEM((_BPW,), jnp.float32),
        pltpu.SemaphoreType.DMA,
    ],
)
def _gather3(a_hbm, ab_hbm, abp_hbm, idx_hbm, oa_hbm, oab_hbm, oabp_hbm,
             ta, tab, tabp, idx_v, va, vab, vabp, sem):
    wid = lax.axis_index("s") * _NC + lax.axis_index("c")
    base = wid * _BPW
    copies = [
        pltpu.async_copy(a_hbm, ta, sem),
        pltpu.async_copy(ab_hbm, tab, sem),
        pltpu.async_copy(abp_hbm, tabp, sem),
        pltpu.async_copy(idx_hbm.at[pl.ds(base, _BPW)], idx_v, sem),
    ]
    for c in copies:
        c.wait()
    def chunk(i, carry):
        for u in range(2):
            sl = pl.ds(i * (2 * _L) + u * _L, _L)
            ids = idx_v[sl]
            va[sl] = plsc.load_gather(ta, [ids])
            vab[sl] = plsc.load_gather(tab, [ids])
            vabp[sl] = plsc.load_gather(tabp, [ids])
        return carry

    lax.fori_loop(0, _NIT // 2, chunk, 0)
    stores = [
        pltpu.async_copy(va, oa_hbm.at[pl.ds(base, _BPW)], sem),
        pltpu.async_copy(vab, oab_hbm.at[pl.ds(base, _BPW)], sem),
        pltpu.async_copy(vabp, oabp_hbm.at[pl.ds(base, _BPW)], sem),
    ]
    for c in stores:
        c.wait()


@jax.jit
def kernel(alphas, alpha_bars, alpha_bars_prev, diffusion_steps):
    a, ab, abp = _gather3(alphas, alpha_bars, alpha_bars_prev,
                          diffusion_steps)
    shape = (-1, 1, 1, 1)
    return a.reshape(shape), ab.reshape(shape), abp.reshape(shape)


# speedup vs baseline: 1.0111x; 1.0111x over previous
import functools

import jax
import jax.numpy as jnp
from jax import lax
from jax.experimental import pallas as pl
from jax.experimental.pallas import tpu as pltpu
from jax.experimental.pallas import tpu_sc as plsc

_T = 1000
_B = 16384
_L = 16

_info = plsc.get_sparse_core_info()
_NC = _info.num_cores
_NS = _info.num_subcores
_NW = _NC * _NS
_BPW = _B // _NW
_NIT = _BPW // _L

_mesh = plsc.VectorSubcoreMesh(core_axis_name="c", subcore_axis_name="s")


@functools.partial(
    pl.kernel,
    mesh=_mesh,
    out_type=(
        jax.ShapeDtypeStruct((_B,), jnp.float32),
        jax.ShapeDtypeStruct((_B,), jnp.float32),
        jax.ShapeDtypeStruct((_B,), jnp.float32),
    ),
    compiler_params=pltpu.CompilerParams(
        needs_layout_passes=False,
        disable_bounds_checks=True,
        disable_semaphore_checks=True,
        skip_device_barrier=True,
    ),
    scratch_types=[
        pltpu.VMEM((_T,), jnp.float32),
        pltpu.VMEM((_T,), jnp.float32),
        pltpu.VMEM((_T,), jnp.float32),
        pltpu.VMEM((_BPW,), jnp.int32),
        pltpu.VMEM((_BPW,), jnp.float32),
        pltpu.VMEM((_BPW,), jnp.float32),
        pltpu.VMEM((_BPW,), jnp.float32),
        pltpu.SemaphoreType.DMA,
    ],
)
def _gather3(a_hbm, ab_hbm, abp_hbm, idx_hbm, oa_hbm, oab_hbm, oabp_hbm,
             ta, tab, tabp, idx_v, va, vab, vabp, sem):
    wid = lax.axis_index("s") * _NC + lax.axis_index("c")
    base = wid * _BPW
    copies = [
        pltpu.async_copy(a_hbm, ta, sem),
        pltpu.async_copy(ab_hbm, tab, sem),
        pltpu.async_copy(abp_hbm, tabp, sem),
        pltpu.async_copy(idx_hbm.at[pl.ds(base, _BPW)], idx_v, sem),
    ]
    for c in copies:
        c.wait()
    @plsc.parallel_loop(0, _NIT, 1, unroll=4)
    def _(i):
        sl = pl.ds(i * _L, _L)
        ids = idx_v[sl]
        va[sl] = plsc.load_gather(ta, [ids])
        vab[sl] = plsc.load_gather(tab, [ids])
        vabp[sl] = plsc.load_gather(tabp, [ids])
    stores = [
        pltpu.async_copy(va, oa_hbm.at[pl.ds(base, _BPW)], sem),
        pltpu.async_copy(vab, oab_hbm.at[pl.ds(base, _BPW)], sem),
        pltpu.async_copy(vabp, oabp_hbm.at[pl.ds(base, _BPW)], sem),
    ]
    for c in stores:
        c.wait()


@jax.jit
def kernel(alphas, alpha_bars, alpha_bars_prev, diffusion_steps):
    a, ab, abp = _gather3(alphas, alpha_bars, alpha_bars_prev,
                          diffusion_steps)
    shape = (-1, 1, 1, 1)
    return a.reshape(shape), ab.reshape(shape), abp.reshape(shape)
